# baseline (device time: 98237 ns/iter reference)
import jax
import jax.numpy as jnp
from jax import lax
from jax.experimental import pallas as pl
from jax.experimental.pallas import tpu as pltpu

N_DEV = 4
SQ = 1024
DM = 1024
HQ = 8
HW = 4
DH = 128
BLK = 256
WIN = 512
SCALE = 0.08838834764831843
LOG2E = 1.4426950408889634
WIN0 = (0, 128, 384, 512)


def _body(x_ref, wqta_ref, wqtb_ref, woa_ref, wob_ref, kext_ref, vext_ref,
          out_ref, wq_cw, wq_ccw, wo_cw, wo_ccw, xb, kland, vland,
          wqcw_s, wqcw_r, wocw_s, wocw_r,
          wqccw_s, wqccw_r, woccw_s, woccw_r, ksem, vsem):
    i = lax.axis_index("i")
    right = lax.rem(i + 1, N_DEV)
    left = lax.rem(i + 3, N_DEV)

    def load_kv(w):
        s, half = divmod(w, 2)
        g = lax.rem(i + (N_DEV - s if half == 0 else s), N_DEV)
        buf = w % 2
        copies = []
        for h in range(HW):
            hh = g * HQ + HW * half + h
            ck = pltpu.make_async_copy(
                kext_ref.at[i, :, hh, :], kland.at[buf, h], ksem.at[buf, h])
            cv = pltpu.make_async_copy(
                vext_ref.at[i, :, hh, :], vland.at[buf, h], vsem.at[buf, h])
            ck.start()
            cv.start()
            copies += [ck, cv]
        return copies

    def wait_kv(copies):
        for c in copies:
            c.wait()

    ld = [None] * 8
    ld[0] = load_kv(0)
    ld[1] = load_kv(1)

    barrier = pltpu.get_barrier_semaphore()
    for nbr in (left, right):
        pl.semaphore_signal(barrier, inc=1, device_id=(nbr,),
                            device_id_type=pl.DeviceIdType.MESH)
    pl.semaphore_wait(barrier, 2)

    xb[...] = x_ref[0].astype(jnp.bfloat16)
    wq_cw[0] = wqta_ref[...]
    wq_ccw[0] = wqtb_ref[...]
    wo_cw[0] = woa_ref[...]
    wo_ccw[0] = wob_ref[...]

    def fwd(buf, h, ss, rr, dev):
        d = pltpu.make_async_remote_copy(
            src_ref=buf.at[h], dst_ref=buf.at[h + 1],
            send_sem=ss.at[h], recv_sem=rr.at[h],
            device_id=(dev,), device_id_type=pl.DeviceIdType.MESH)
        d.start()
        return d

    def make_mbias(r):
        qi = BLK * r + lax.broadcasted_iota(jnp.int32, (BLK, WIN), 0)
        kj = WIN0[r] + lax.broadcasted_iota(jnp.int32, (BLK, WIN), 1)
        return jnp.where(jnp.abs(qi - kj) <= 128, 0.0, -1e30
                         ).astype(jnp.float32)

    mbias_all = [make_mbias(r) for r in range(SQ // BLK)]

    def phase1(w):
        s, half = divmod(w, 2)
        wq = (wq_cw if half == 0 else wq_ccw)[s]
        buf = w % 2
        q = lax.dot_general(xb[...], wq, (((1,), (1,)), ((), ())),
                            preferred_element_type=jnp.float32
                            ).astype(jnp.bfloat16)
        ctxs = []
        for rl in range(SQ // BLK):
            w0 = WIN0[rl]
            mbias = mbias_all[rl]
            heads = []
            for h in range(HW):
                qb = q[BLK * rl:BLK * (rl + 1), DH * h:DH * (h + 1)]
                kb = kland[buf, h, w0:w0 + WIN, :].astype(jnp.bfloat16)
                vb = vland[buf, h, w0:w0 + WIN, :].astype(jnp.bfloat16)
                sc = lax.dot_general(
                    qb, kb, (((1,), (1,)), ((), ())),
                    preferred_element_type=jnp.float32)
                e = jnp.exp2(sc + mbias)
                rs_inv = 1.0 / jnp.sum(e, axis=-1, keepdims=True)
                ctx = jnp.dot(e.astype(jnp.bfloat16), vb,
                              preferred_element_type=jnp.float32)
                heads.append((ctx * rs_inv).astype(jnp.bfloat16))
            ctxs.append(jnp.concatenate(heads, axis=1))
        return ctxs

    def phase2(w, ctxs):
        s, half = divmod(w, 2)
        wo = (wo_cw if half == 0 else wo_ccw)[s]
        for rl in range(SQ // BLK):
            acc = jnp.dot(ctxs[rl], wo,
                          preferred_element_type=jnp.float32)
            sl = (0, pl.ds(BLK * rl, BLK), slice(None))
            if w == 0:
                out_ref[sl] = acc
            else:
                out_ref[sl] = out_ref[sl] + acc

    h0_wq_cw = fwd(wq_cw, 0, wqcw_s, wqcw_r, right)
    h0_wo_cw = fwd(wo_cw, 0, wocw_s, wocw_r, right)
    h0_wq_ccw = fwd(wq_ccw, 0, wqccw_s, wqccw_r, left)
    h0_wo_ccw = fwd(wo_ccw, 0, woccw_s, woccw_r, left)

    wait_kv(ld[0])
    phase2(0, phase1(0))
    ld[2] = load_kv(2)
    wait_kv(ld[1])
    phase2(1, phase1(1))
    ld[3] = load_kv(3)

    h0_wq_cw.wait()
    h1_wq_cw = fwd(wq_cw, 1, wqcw_s, wqcw_r, right)
    h0_wo_cw.wait()
    h1_wo_cw = fwd(wo_cw, 1, wocw_s, wocw_r, right)
    wait_kv(ld[2])
    phase2(2, phase1(2))
    ld[4] = load_kv(4)

    h0_wq_ccw.wait()
    h1_wq_ccw = fwd(wq_ccw, 1, wqccw_s, wqccw_r, left)
    h0_wo_ccw.wait()
    h1_wo_ccw = fwd(wo_ccw, 1, woccw_s, woccw_r, left)
    wait_kv(ld[3])
    phase2(3, phase1(3))
    ld[5] = load_kv(5)

    h1_wq_cw.wait()
    h2_wq_cw = fwd(wq_cw, 2, wqcw_s, wqcw_r, right)
    h1_wo_cw.wait()
    h2_wo_cw = fwd(wo_cw, 2, wocw_s, wocw_r, right)
    wait_kv(ld[4])
    phase2(4, phase1(4))
    ld[6] = load_kv(6)

    h1_wq_ccw.wait()
    h2_wq_ccw = fwd(wq_ccw, 2, wqccw_s, wqccw_r, left)
    h1_wo_ccw.wait()
    h2_wo_ccw = fwd(wo_ccw, 2, woccw_s, woccw_r, left)
    wait_kv(ld[5])
    phase2(5, phase1(5))
    ld[7] = load_kv(7)

    h2_wq_cw.wait()
    wait_kv(ld[6])
    c6 = phase1(6)
    h2_wo_cw.wait()
    phase2(6, c6)

    h2_wq_ccw.wait()
    wait_kv(ld[7])
    c7 = phase1(7)
    h2_wo_ccw.wait()
    phase2(7, c7)


def kernel(x, Wq, K_ext, V_ext, Wo):
    wqs = (Wq * (SCALE * LOG2E)).astype(jnp.bfloat16)
    wqta = wqs[:, :DM // 2].T
    wqtb = wqs[:, DM // 2:].T
    wo16 = Wo.astype(jnp.bfloat16)
    woa = wo16[:DM // 2, :]
    wob = wo16[DM // 2:, :]

    wslab = (N_DEV, DM // 2, DM)
    return pl.pallas_call(
        _body,
        out_shape=jax.ShapeDtypeStruct((1, SQ, DM), jnp.float32),
        in_specs=[
            pl.BlockSpec(memory_space=pltpu.VMEM),
            pl.BlockSpec(memory_space=pltpu.VMEM),
            pl.BlockSpec(memory_space=pltpu.VMEM),
            pl.BlockSpec(memory_space=pltpu.VMEM),
            pl.BlockSpec(memory_space=pltpu.VMEM),
            pl.BlockSpec(memory_space=pl.ANY),
            pl.BlockSpec(memory_space=pl.ANY),
        ],
        out_specs=pl.BlockSpec(memory_space=pltpu.VMEM),
        scratch_shapes=[
            pltpu.VMEM(wslab, jnp.bfloat16),
            pltpu.VMEM(wslab, jnp.bfloat16),
            pltpu.VMEM(wslab, jnp.bfloat16),
            pltpu.VMEM(wslab, jnp.bfloat16),
            pltpu.VMEM((SQ, DM), jnp.bfloat16),
            pltpu.VMEM((2, HW, SQ, DH), jnp.float32),
            pltpu.VMEM((2, HW, SQ, DH), jnp.float32),
            pltpu.SemaphoreType.DMA((N_DEV - 1,)),
            pltpu.SemaphoreType.DMA((N_DEV - 1,)),
            pltpu.SemaphoreType.DMA((N_DEV - 1,)),
            pltpu.SemaphoreType.DMA((N_DEV - 1,)),
            pltpu.SemaphoreType.DMA((N_DEV - 1,)),
            pltpu.SemaphoreType.DMA((N_DEV - 1,)),
            pltpu.SemaphoreType.DMA((N_DEV - 1,)),
            pltpu.SemaphoreType.DMA((N_DEV - 1,)),
            pltpu.SemaphoreType.DMA((2, HW)),
            pltpu.SemaphoreType.DMA((2, HW)),
        ],
        compiler_params=pltpu.CompilerParams(
            collective_id=0, vmem_limit_bytes=48 * 1024 * 1024),
    )(x, wqta, wqtb, woa, wob, K_ext, V_ext)


# device time: 97421 ns/iter; 1.0084x vs baseline; 1.0084x over previous
import jax
import jax.numpy as jnp
from jax import lax
from jax.experimental import pallas as pl
from jax.experimental.pallas import tpu as pltpu

N_DEV = 4
SQ = 1024
DM = 1024
HQ = 8
HW = 4
DH = 128
BLK = 256
WIN = 512
SCALE = 0.08838834764831843
LOG2E = 1.4426950408889634
WIN0 = (0, 128, 384, 512)


def _body(x_ref, wqta_ref, wqtb_ref, woa_ref, wob_ref, kext_ref, vext_ref,
          out_ref, wq_cw, wq_ccw, wo_cw, wo_ccw, xb, kland, vland,
          wqcw_s, wqcw_r, wocw_s, wocw_r,
          wqccw_s, wqccw_r, woccw_s, woccw_r, ksem, vsem):
    i = lax.axis_index("i")
    right = lax.rem(i + 1, N_DEV)
    left = lax.rem(i + 3, N_DEV)

    def load_kv(w):
        s, half = divmod(w, 2)
        g = lax.rem(i + (N_DEV - s if half == 0 else s), N_DEV)
        buf = w % 2
        copies = []
        for h in range(HW):
            hh = g * HQ + HW * half + h
            ck = pltpu.make_async_copy(
                kext_ref.at[i, :, hh, :], kland.at[buf, h], ksem.at[buf, h])
            cv = pltpu.make_async_copy(
                vext_ref.at[i, :, hh, :], vland.at[buf, h], vsem.at[buf, h])
            ck.start()
            cv.start()
            copies += [ck, cv]
        return copies

    def wait_kv(copies):
        for c in copies:
            c.wait()

    ld = [None] * 8
    ld[0] = load_kv(0)
    ld[1] = load_kv(1)

    barrier = pltpu.get_barrier_semaphore()
    for nbr in (left, right):
        pl.semaphore_signal(barrier, inc=1, device_id=(nbr,),
                            device_id_type=pl.DeviceIdType.MESH)
    pl.semaphore_wait(barrier, 2)

    def fwd(buf, h, ss, rr, dev, own):
        d = pltpu.make_async_remote_copy(
            src_ref=own if h == 0 else buf.at[h - 1], dst_ref=buf.at[h],
            send_sem=ss.at[h], recv_sem=rr.at[h],
            device_id=(dev,), device_id_type=pl.DeviceIdType.MESH)
        d.start()
        return d

    h0_wq_cw = fwd(wq_cw, 0, wqcw_s, wqcw_r, right, wqta_ref)
    h0_wo_cw = fwd(wo_cw, 0, wocw_s, wocw_r, right, woa_ref)
    h0_wq_ccw = fwd(wq_ccw, 0, wqccw_s, wqccw_r, left, wqtb_ref)
    h0_wo_ccw = fwd(wo_ccw, 0, woccw_s, woccw_r, left, wob_ref)

    xb[...] = x_ref[0].astype(jnp.bfloat16)

    def make_mbias(r):
        qi = BLK * r + lax.broadcasted_iota(jnp.int32, (BLK, WIN), 0)
        kj = WIN0[r] + lax.broadcasted_iota(jnp.int32, (BLK, WIN), 1)
        return jnp.where(jnp.abs(qi - kj) <= 128, 0.0, -1e30
                         ).astype(jnp.float32)

    mbias_all = [make_mbias(r) for r in range(SQ // BLK)]

    def phase1(w):
        s, half = divmod(w, 2)
        if s == 0:
            wq = (wqta_ref if half == 0 else wqtb_ref)[...]
        else:
            wq = (wq_cw if half == 0 else wq_ccw)[s - 1]
        buf = w % 2
        q = lax.dot_general(xb[...], wq, (((1,), (1,)), ((), ())),
                            preferred_element_type=jnp.float32
                            ).astype(jnp.bfloat16)
        ctxs = []
        for rl in range(SQ // BLK):
            w0 = WIN0[rl]
            mbias = mbias_all[rl]
            heads = []
            for h in range(HW):
                qb = q[BLK * rl:BLK * (rl + 1), DH * h:DH * (h + 1)]
                kb = kland[buf, h, w0:w0 + WIN, :].astype(jnp.bfloat16)
                vb = vland[buf, h, w0:w0 + WIN, :].astype(jnp.bfloat16)
                sc = lax.dot_general(
                    qb, kb, (((1,), (1,)), ((), ())),
                    preferred_element_type=jnp.float32)
                e = jnp.exp2(sc + mbias)
                rs_inv = 1.0 / jnp.sum(e, axis=-1, keepdims=True)
                ctx = jnp.dot(e.astype(jnp.bfloat16), vb,
                              preferred_element_type=jnp.float32)
                heads.append((ctx * rs_inv).astype(jnp.bfloat16))
            ctxs.append(jnp.concatenate(heads, axis=1))
        return ctxs

    def phase2(w, ctxs):
        s, half = divmod(w, 2)
        if s == 0:
            wo = (woa_ref if half == 0 else wob_ref)[...]
        else:
            wo = (wo_cw if half == 0 else wo_ccw)[s - 1]
        for rl in range(SQ // BLK):
            acc = jnp.dot(ctxs[rl], wo,
                          preferred_element_type=jnp.float32)
            sl = (0, pl.ds(BLK * rl, BLK), slice(None))
            if w == 0:
                out_ref[sl] = acc
            else:
                out_ref[sl] = out_ref[sl] + acc

    wait_kv(ld[0])
    phase2(0, phase1(0))
    ld[2] = load_kv(2)
    wait_kv(ld[1])
    phase2(1, phase1(1))
    ld[3] = load_kv(3)

    h0_wq_cw.wait()
    h1_wq_cw = fwd(wq_cw, 1, wqcw_s, wqcw_r, right, wqta_ref)
    h0_wo_cw.wait()
    h1_wo_cw = fwd(wo_cw, 1, wocw_s, wocw_r, right, woa_ref)
    wait_kv(ld[2])
    phase2(2, phase1(2))
    ld[4] = load_kv(4)

    h0_wq_ccw.wait()
    h1_wq_ccw = fwd(wq_ccw, 1, wqccw_s, wqccw_r, left, wqtb_ref)
    h0_wo_ccw.wait()
    h1_wo_ccw = fwd(wo_ccw, 1, woccw_s, woccw_r, left, wob_ref)
    wait_kv(ld[3])
    phase2(3, phase1(3))
    ld[5] = load_kv(5)

    h1_wq_cw.wait()
    h2_wq_cw = fwd(wq_cw, 2, wqcw_s, wqcw_r, right, wqta_ref)
    h1_wo_cw.wait()
    h2_wo_cw = fwd(wo_cw, 2, wocw_s, wocw_r, right, woa_ref)
    wait_kv(ld[4])
    phase2(4, phase1(4))
    ld[6] = load_kv(6)

    h1_wq_ccw.wait()
    h2_wq_ccw = fwd(wq_ccw, 2, wqccw_s, wqccw_r, left, wqtb_ref)
    h1_wo_ccw.wait()
    h2_wo_ccw = fwd(wo_ccw, 2, woccw_s, woccw_r, left, wob_ref)
    wait_kv(ld[5])
    phase2(5, phase1(5))
    ld[7] = load_kv(7)

    h2_wq_cw.wait()
    wait_kv(ld[6])
    c6 = phase1(6)
    h2_wo_cw.wait()
    phase2(6, c6)

    h2_wq_ccw.wait()
    wait_kv(ld[7])
    c7 = phase1(7)
    h2_wo_ccw.wait()
    phase2(7, c7)


def kernel(x, Wq, K_ext, V_ext, Wo):
    wqs = (Wq * (SCALE * LOG2E)).astype(jnp.bfloat16)
    wqta = wqs[:, :DM // 2].T
    wqtb = wqs[:, DM // 2:].T
    wo16 = Wo.astype(jnp.bfloat16)
    woa = wo16[:DM // 2, :]
    wob = wo16[DM // 2:, :]

    wslab = (N_DEV - 1, DM // 2, DM)
    return pl.pallas_call(
        _body,
        out_shape=jax.ShapeDtypeStruct((1, SQ, DM), jnp.float32),
        in_specs=[
            pl.BlockSpec(memory_space=pltpu.VMEM),
            pl.BlockSpec(memory_space=pltpu.VMEM),
            pl.BlockSpec(memory_space=pltpu.VMEM),
            pl.BlockSpec(memory_space=pltpu.VMEM),
            pl.BlockSpec(memory_space=pltpu.VMEM),
            pl.BlockSpec(memory_space=pl.ANY),
            pl.BlockSpec(memory_space=pl.ANY),
        ],
        out_specs=pl.BlockSpec(memory_space=pltpu.VMEM),
        scratch_shapes=[
            pltpu.VMEM(wslab, jnp.bfloat16),
            pltpu.VMEM(wslab, jnp.bfloat16),
            pltpu.VMEM(wslab, jnp.bfloat16),
            pltpu.VMEM(wslab, jnp.bfloat16),
            pltpu.VMEM((SQ, DM), jnp.bfloat16),
            pltpu.VMEM((2, HW, SQ, DH), jnp.float32),
            pltpu.VMEM((2, HW, SQ, DH), jnp.float32),
            pltpu.SemaphoreType.DMA((N_DEV - 1,)),
            pltpu.SemaphoreType.DMA((N_DEV - 1,)),
            pltpu.SemaphoreType.DMA((N_DEV - 1,)),
            pltpu.SemaphoreType.DMA((N_DEV - 1,)),
            pltpu.SemaphoreType.DMA((N_DEV - 1,)),
            pltpu.SemaphoreType.DMA((N_DEV - 1,)),
            pltpu.SemaphoreType.DMA((N_DEV - 1,)),
            pltpu.SemaphoreType.DMA((N_DEV - 1,)),
            pltpu.SemaphoreType.DMA((2, HW)),
            pltpu.SemaphoreType.DMA((2, HW)),
        ],
        compiler_params=pltpu.CompilerParams(
            collective_id=0, vmem_limit_bytes=48 * 1024 * 1024),
    )(x, wqta, wqtb, woa, wob, K_ext, V_ext)
